# initial kernel scaffold (unmeasured)
import jax
import jax.numpy as jnp
from jax import lax
from jax.experimental import pallas as pl
from jax.experimental.pallas import tpu as pltpu

N_DEV = 4
_GELU_C = 0.7978845608028654


def kernel(x, w_mat):
    m_per, k = x.shape
    _, n = w_mat.shape
    n_per = n // N_DEV

    def body(x_ref, w_ref, out_ref, send_buf, recv_buf, send_sems, recv_sems):
        my = lax.axis_index("i")

        barrier = pltpu.get_barrier_semaphore()
        for d in range(1, N_DEV):
            pl.semaphore_signal(
                barrier,
                inc=1,
                device_id=((my + d) % N_DEV,),
                device_id_type=pl.DeviceIdType.MESH,
            )
        pl.semaphore_wait(barrier, N_DEV - 1)

        x_val = x_ref[...]

        def chunk(t):
            y = jnp.dot(
                x_val,
                w_ref[:, pl.ds(t * n_per, n_per)],
                preferred_element_type=jnp.float32,
            )
            return 0.5 * y * (1.0 + jnp.tanh(_GELU_C * (y + 0.044715 * y * y * y)))

        rdmas = {}
        for d in (2, 1, 3):
            t = (my + d) % N_DEV
            send_buf[d - 1, :, :] = chunk(t).astype(jnp.bfloat16)
            rdma = pltpu.make_async_remote_copy(
                src_ref=send_buf.at[d - 1],
                dst_ref=recv_buf.at[d - 1],
                send_sem=send_sems.at[d - 1],
                recv_sem=recv_sems.at[d - 1],
                device_id=(t,),
                device_id_type=pl.DeviceIdType.MESH,
            )
            rdma.start()
            rdmas[d] = rdma

        out_ref[pl.ds(my * m_per, m_per), :] = chunk(my)

        for d in (1, 3, 2):
            rdmas[d].wait_recv()
            src = (my - d) % N_DEV
            out_ref[pl.ds(src * m_per, m_per), :] = recv_buf[d - 1].astype(
                jnp.float32
            )

        for d in (1, 2, 3):
            rdmas[d].wait_send()

    out_shape = jax.ShapeDtypeStruct((N_DEV * m_per, n_per), jnp.float32)
    return pl.pallas_call(
        body,
        out_shape=out_shape,
        in_specs=[
            pl.BlockSpec(memory_space=pltpu.VMEM),
            pl.BlockSpec(memory_space=pltpu.VMEM),
        ],
        out_specs=pl.BlockSpec(memory_space=pltpu.VMEM),
        scratch_shapes=[
            pltpu.VMEM((N_DEV - 1, m_per, n_per), jnp.bfloat16),
            pltpu.VMEM((N_DEV - 1, m_per, n_per), jnp.bfloat16),
            pltpu.SemaphoreType.DMA((N_DEV - 1,)),
            pltpu.SemaphoreType.DMA((N_DEV - 1,)),
        ],
        compiler_params=pltpu.CompilerParams(collective_id=0),
    )(x, w_mat)


# baseline (device time: 66459 ns/iter reference)
import jax
import jax.numpy as jnp
from jax import lax
from jax.experimental import pallas as pl
from jax.experimental.pallas import tpu as pltpu

N_DEV = 4
_GELU_C = 0.7978845608028654


def kernel(x, w_mat):
    m_per, k = x.shape
    _, n = w_mat.shape
    n_per = n // N_DEV
    x = x.astype(jnp.bfloat16)
    w_mat = w_mat.astype(jnp.bfloat16)

    def body(x_ref, w_ref, out_ref, send_buf, recv_buf, send_sems, recv_sems):
        my = lax.axis_index("i")

        barrier = pltpu.get_barrier_semaphore()
        for d in range(1, N_DEV):
            pl.semaphore_signal(
                barrier,
                inc=1,
                device_id=((my + d) % N_DEV,),
                device_id_type=pl.DeviceIdType.MESH,
            )
        pl.semaphore_wait(barrier, N_DEV - 1)

        x_val = x_ref[...]

        def chunk(t):
            y = jnp.dot(
                x_val,
                w_ref[:, pl.ds(t * n_per, n_per)],
                preferred_element_type=jnp.float32,
            )
            return 0.5 * y * (1.0 + jnp.tanh(_GELU_C * (y + 0.044715 * y * y * y)))

        rdmas = {}
        for d in (2, 1, 3):
            t = (my + d) % N_DEV
            send_buf[d - 1, :, :] = chunk(t).astype(jnp.bfloat16)
            rdma = pltpu.make_async_remote_copy(
                src_ref=send_buf.at[d - 1],
                dst_ref=recv_buf.at[d - 1],
                send_sem=send_sems.at[d - 1],
                recv_sem=recv_sems.at[d - 1],
                device_id=(t,),
                device_id_type=pl.DeviceIdType.MESH,
            )
            rdma.start()
            rdmas[d] = rdma

        out_ref[pl.ds(my * m_per, m_per), :] = chunk(my)

        for d in (1, 3, 2):
            rdmas[d].wait_recv()
            src = (my - d) % N_DEV
            out_ref[pl.ds(src * m_per, m_per), :] = recv_buf[d - 1].astype(
                jnp.float32
            )

        for d in (1, 2, 3):
            rdmas[d].wait_send()

    out_shape = jax.ShapeDtypeStruct((N_DEV * m_per, n_per), jnp.float32)
    return pl.pallas_call(
        body,
        out_shape=out_shape,
        in_specs=[
            pl.BlockSpec(memory_space=pltpu.VMEM),
            pl.BlockSpec(memory_space=pltpu.VMEM),
        ],
        out_specs=pl.BlockSpec(memory_space=pltpu.VMEM),
        scratch_shapes=[
            pltpu.VMEM((N_DEV - 1, m_per, n_per), jnp.bfloat16),
            pltpu.VMEM((N_DEV - 1, m_per, n_per), jnp.bfloat16),
            pltpu.SemaphoreType.DMA((N_DEV - 1,)),
            pltpu.SemaphoreType.DMA((N_DEV - 1,)),
        ],
        compiler_params=pltpu.CompilerParams(collective_id=0),
    )(x, w_mat)


# device time: 51225 ns/iter; 1.2974x vs baseline; 1.2974x over previous
import jax
import jax.numpy as jnp
from jax import lax
from jax.experimental import pallas as pl
from jax.experimental.pallas import tpu as pltpu

N_DEV = 4
_GELU_C = 0.7978845608028654
_XT = 4


def _gelu(y):
    return 0.5 * y * (1.0 + jnp.tanh(_GELU_C * (y + 0.044715 * y * y * y)))


def kernel(x, w_mat):
    m_per, k = x.shape
    _, n = w_mat.shape
    n_per = n // N_DEV
    xt = m_per // _XT

    def body(
        x_hbm,
        w_hbm,
        out_ref,
        x_land,
        x_bf,
        w_land,
        w_bf,
        send_buf,
        recv_buf,
        x_sems,
        w_sems,
        send_sems,
        recv_sems,
    ):
        my = lax.axis_index("i")

        offsets = (2, 1, 3, 0)

        def w_copy(slot, t):
            return pltpu.make_async_copy(
                w_hbm.at[:, pl.ds(t * n_per, n_per)],
                w_land.at[slot],
                w_sems.at[slot],
            )

        x_copies = []
        for i in range(2):
            c = pltpu.make_async_copy(
                x_hbm.at[pl.ds(i * xt, xt), :], x_land.at[i], x_sems.at[i]
            )
            c.start()
            x_copies.append(c)
        wc_prev = w_copy(0, (my + offsets[0]) % N_DEV)
        wc_prev.start()

        barrier = pltpu.get_barrier_semaphore()
        for d in range(1, N_DEV):
            pl.semaphore_signal(
                barrier,
                inc=1,
                device_id=((my + d) % N_DEV,),
                device_id_type=pl.DeviceIdType.MESH,
            )
        pl.semaphore_wait(barrier, N_DEV - 1)

        for i in range(_XT):
            x_copies[i].wait()
            if i + 2 < _XT:
                c = pltpu.make_async_copy(
                    x_hbm.at[pl.ds((i + 2) * xt, xt), :],
                    x_land.at[i % 2],
                    x_sems.at[i % 2],
                )
                x_copies.append(c)
            x_bf[pl.ds(i * xt, xt), :] = x_land[i % 2].astype(jnp.bfloat16)
            if i + 2 < _XT:
                x_copies[i + 2].start()

        x_val = x_bf[...]

        rdmas = {}
        for j, d in enumerate(offsets):
            wc_prev.wait()
            slot = j % 2
            if j + 1 < N_DEV:
                nxt = w_copy((j + 1) % 2, (my + offsets[j + 1]) % N_DEV)
                nxt.start()
            w_bf[...] = w_land[slot].astype(jnp.bfloat16)
            y = _gelu(
                jnp.dot(x_val, w_bf[...], preferred_element_type=jnp.float32)
            )
            if d == 0:
                out_ref[pl.ds(my * m_per, m_per), :] = y
            else:
                send_buf[d - 1, :, :] = y.astype(jnp.bfloat16)
                rdma = pltpu.make_async_remote_copy(
                    src_ref=send_buf.at[d - 1],
                    dst_ref=recv_buf.at[d - 1],
                    send_sem=send_sems.at[d - 1],
                    recv_sem=recv_sems.at[d - 1],
                    device_id=((my + d) % N_DEV,),
                    device_id_type=pl.DeviceIdType.MESH,
                )
                rdma.start()
                rdmas[d] = rdma
            if j + 1 < N_DEV:
                wc_prev = nxt

        for d in (1, 3, 2):
            rdmas[d].wait_recv()
            src = (my - d) % N_DEV
            out_ref[pl.ds(src * m_per, m_per), :] = recv_buf[d - 1].astype(
                jnp.float32
            )

        for d in (1, 2, 3):
            rdmas[d].wait_send()

    out_shape = jax.ShapeDtypeStruct((N_DEV * m_per, n_per), jnp.float32)
    return pl.pallas_call(
        body,
        out_shape=out_shape,
        in_specs=[
            pl.BlockSpec(memory_space=pl.ANY),
            pl.BlockSpec(memory_space=pl.ANY),
        ],
        out_specs=pl.BlockSpec(memory_space=pltpu.VMEM),
        scratch_shapes=[
            pltpu.VMEM((2, xt, k), jnp.float32),
            pltpu.VMEM((m_per, k), jnp.bfloat16),
            pltpu.VMEM((2, k, n_per), jnp.float32),
            pltpu.VMEM((k, n_per), jnp.bfloat16),
            pltpu.VMEM((N_DEV - 1, m_per, n_per), jnp.bfloat16),
            pltpu.VMEM((N_DEV - 1, m_per, n_per), jnp.bfloat16),
            pltpu.SemaphoreType.DMA((2,)),
            pltpu.SemaphoreType.DMA((2,)),
            pltpu.SemaphoreType.DMA((N_DEV - 1,)),
            pltpu.SemaphoreType.DMA((N_DEV - 1,)),
        ],
        compiler_params=pltpu.CompilerParams(
            collective_id=0, vmem_limit_bytes=100 * 1024 * 1024
        ),
    )(x, w_mat)


# device time: 44864 ns/iter; 1.4813x vs baseline; 1.1418x over previous
import jax
import jax.numpy as jnp
from jax import lax
from jax.experimental import pallas as pl
from jax.experimental.pallas import tpu as pltpu

N_DEV = 4
_GELU_C = 0.7978845608028654
_RT = 4
_WH = 2


def _gelu(y):
    return 0.5 * y * (1.0 + jnp.tanh(_GELU_C * (y + 0.044715 * y * y * y)))


def kernel(x, w_mat):
    m_per, k = x.shape
    _, n = w_mat.shape
    n_per = n // N_DEV
    rt = m_per // _RT
    wh = n_per // _WH

    offs = (2, 1, 3, 0)

    def body(
        x_hbm,
        w_hbm,
        out_hbm,
        x_land,
        x_bf,
        w_land,
        w_bf,
        send_buf,
        recv_buf,
        stage,
        x_sems,
        w_sems,
        out_sems,
        send_sems,
        recv_sems,
    ):
        my = lax.axis_index("i")

        def x_copy(i):
            return pltpu.make_async_copy(
                x_hbm.at[pl.ds(i * rt, rt), :],
                x_land.at[i % 2],
                x_sems.at[i % 2],
            )

        def w_copy(j, h):
            t = (my + offs[j]) % N_DEV
            return pltpu.make_async_copy(
                w_hbm.at[:, pl.ds(t * n_per + h * wh, wh)],
                w_land.at[h],
                w_sems.at[h],
            )

        xc = [x_copy(i) for i in range(_RT)]
        wc = {(j, h): w_copy(j, h) for j in range(4) for h in range(_WH)}
        xc[0].start()
        xc[1].start()
        wc[(0, 0)].start()
        wc[(0, 1)].start()

        barrier = pltpu.get_barrier_semaphore()
        for d in range(1, N_DEV):
            pl.semaphore_signal(
                barrier,
                inc=1,
                device_id=((my + d) % N_DEV,),
                device_id_type=pl.DeviceIdType.MESH,
            )
        pl.semaphore_wait(barrier, N_DEV - 1)

        def xwait(i):
            xc[i].wait()
            x_bf[pl.ds(i * rt, rt), :] = x_land[i % 2].astype(jnp.bfloat16)
            if i + 2 < _RT:
                xc[i + 2].start()

        def wwait(j, h):
            wc[(j, h)].wait()
            w_bf[j, :, pl.ds(h * wh, wh)] = w_land[h].astype(jnp.bfloat16)
            if j + 1 < 4:
                wc[(j + 1, h)].start()

        rdmas = {}

        def sub(r, j):
            y = _gelu(
                jnp.dot(
                    x_bf[pl.ds(r * rt, rt), :],
                    w_bf[j],
                    preferred_element_type=jnp.float32,
                )
            )
            if offs[j] == 0:
                stage[0, pl.ds(r * rt, rt), :] = y
            else:
                send_buf[j, pl.ds(r * rt, rt), :] = y.astype(jnp.bfloat16)
                rdma = pltpu.make_async_remote_copy(
                    src_ref=send_buf.at[j, pl.ds(r * rt, rt), :],
                    dst_ref=recv_buf.at[j, pl.ds(r * rt, rt), :],
                    send_sem=send_sems.at[j, r],
                    recv_sem=recv_sems.at[j, r],
                    device_id=((my + offs[j]) % N_DEV,),
                    device_id_type=pl.DeviceIdType.MESH,
                )
                rdma.start()
                rdmas[(r, j)] = rdma

        xwait(0)
        xwait(1)
        wwait(0, 0)
        wwait(0, 1)
        sub(0, 0)
        sub(1, 0)
        xwait(2)
        sub(2, 0)
        xwait(3)
        sub(3, 0)
        for j in (1, 2, 3):
            wwait(j, 0)
            wwait(j, 1)
            for r in range(_RT):
                sub(r, j)

        oc = pltpu.make_async_copy(
            stage.at[0],
            out_hbm.at[pl.ds(my * m_per, m_per), :],
            out_sems.at[0],
        )
        oc.start()
        out_copies = [oc, None]

        slot = 1
        for j in (1, 2, 0):
            for r in range(_RT):
                rdmas[(r, j)].wait_recv()
            src = (my - offs[j]) % N_DEV
            if out_copies[slot] is not None:
                out_copies[slot].wait()
            stage[slot] = recv_buf[j].astype(jnp.float32)
            oc = pltpu.make_async_copy(
                stage.at[slot],
                out_hbm.at[pl.ds(src * m_per, m_per), :],
                out_sems.at[slot],
            )
            oc.start()
            out_copies[slot] = oc
            slot ^= 1

        out_copies[0].wait()
        out_copies[1].wait()
        for rd in rdmas.values():
            rd.wait_send()

    out_shape = jax.ShapeDtypeStruct((N_DEV * m_per, n_per), jnp.float32)
    return pl.pallas_call(
        body,
        out_shape=out_shape,
        in_specs=[
            pl.BlockSpec(memory_space=pl.ANY),
            pl.BlockSpec(memory_space=pl.ANY),
        ],
        out_specs=pl.BlockSpec(memory_space=pl.ANY),
        scratch_shapes=[
            pltpu.VMEM((2, rt, k), jnp.float32),
            pltpu.VMEM((m_per, k), jnp.bfloat16),
            pltpu.VMEM((2, k, wh), jnp.float32),
            pltpu.VMEM((4, k, n_per), jnp.bfloat16),
            pltpu.VMEM((3, m_per, n_per), jnp.bfloat16),
            pltpu.VMEM((3, m_per, n_per), jnp.bfloat16),
            pltpu.VMEM((2, m_per, n_per), jnp.float32),
            pltpu.SemaphoreType.DMA((2,)),
            pltpu.SemaphoreType.DMA((2,)),
            pltpu.SemaphoreType.DMA((2,)),
            pltpu.SemaphoreType.DMA((3, _RT)),
            pltpu.SemaphoreType.DMA((3, _RT)),
        ],
        compiler_params=pltpu.CompilerParams(
            collective_id=0, vmem_limit_bytes=100 * 1024 * 1024
        ),
    )(x, w_mat)


# device time: 44005 ns/iter; 1.5103x vs baseline; 1.0195x over previous
import jax
import jax.numpy as jnp
from jax import lax
from jax.experimental import pallas as pl
from jax.experimental.pallas import tpu as pltpu

N_DEV = 4
_GELU_C = 0.7978845608028654
_RT = 4
_WH = 2


def _gelu(y):
    return 0.5 * y * (1.0 + jnp.tanh(_GELU_C * (y + 0.044715 * y * y * y)))


def kernel(x, w_mat):
    m_per, k = x.shape
    _, n = w_mat.shape
    n_per = n // N_DEV
    rt = m_per // _RT
    wh = n_per // _WH

    offs = (2, 1, 3, 0)

    def body(
        x_hbm,
        w_hbm,
        out_hbm,
        x_land,
        x_bf,
        w_land,
        w_bf,
        send_buf,
        recv_buf,
        stage,
        x_sems,
        w_sems,
        out_sems,
        send_sems,
        recv_sems,
    ):
        my = lax.axis_index("i")

        def x_copy(i):
            return pltpu.make_async_copy(
                x_hbm.at[pl.ds(i * rt, rt), :],
                x_land.at[i % 2],
                x_sems.at[i % 2],
            )

        def w_copy(j, h):
            t = (my + offs[j]) % N_DEV
            return pltpu.make_async_copy(
                w_hbm.at[:, pl.ds(t * n_per + h * wh, wh)],
                w_land.at[h],
                w_sems.at[h],
            )

        xc = [x_copy(i) for i in range(_RT)]
        wc = {(j, h): w_copy(j, h) for j in range(4) for h in range(_WH)}
        xc[0].start()
        wc[(0, 0)].start()
        xc[1].start()
        wc[(0, 1)].start()

        barrier = pltpu.get_barrier_semaphore()
        for d in range(1, N_DEV):
            pl.semaphore_signal(
                barrier,
                inc=1,
                device_id=((my + d) % N_DEV,),
                device_id_type=pl.DeviceIdType.MESH,
            )
        pl.semaphore_wait(barrier, N_DEV - 1)

        def xwait(i):
            xc[i].wait()
            x_bf[pl.ds(i * rt, rt), :] = x_land[i % 2].astype(jnp.bfloat16)
            if i + 2 < _RT:
                xc[i + 2].start()

        def wwait(j, h):
            wc[(j, h)].wait()
            w_bf[j, :, pl.ds(h * wh, wh)] = w_land[h].astype(jnp.bfloat16)
            if j + 1 < 4:
                wc[(j + 1, h)].start()

        rdmas = {}

        def emit_rdma(j, rows, cols, key):
            rdma = pltpu.make_async_remote_copy(
                src_ref=send_buf.at[j, rows, cols],
                dst_ref=recv_buf.at[j, rows, cols],
                send_sem=send_sems.at[key],
                recv_sem=recv_sems.at[key],
                device_id=((my + offs[j]) % N_DEV,),
                device_id_type=pl.DeviceIdType.MESH,
            )
            rdma.start()
            rdmas[key] = rdma

        def subq(r, j, h):
            rows = pl.ds(r * rt, rt)
            cols = pl.ds(h * wh, wh)
            y = _gelu(
                jnp.dot(
                    x_bf[rows, :],
                    w_bf[j, :, cols],
                    preferred_element_type=jnp.float32,
                )
            )
            send_buf[j, rows, cols] = y.astype(jnp.bfloat16)
            emit_rdma(j, rows, cols, (j, r, h))

        def sub(r, j):
            rows = pl.ds(r * rt, rt)
            y = _gelu(
                jnp.dot(
                    x_bf[rows, :],
                    w_bf[j],
                    preferred_element_type=jnp.float32,
                )
            )
            if offs[j] == 0:
                stage[0, rows, :] = y
            else:
                send_buf[j, rows, :] = y.astype(jnp.bfloat16)
                emit_rdma(j, rows, slice(None), (j, r, 0))

        xwait(0)
        wwait(0, 0)
        subq(0, 0, 0)
        xwait(1)
        subq(1, 0, 0)
        wwait(0, 1)
        subq(0, 0, 1)
        subq(1, 0, 1)
        xwait(2)
        subq(2, 0, 0)
        subq(2, 0, 1)
        xwait(3)
        subq(3, 0, 0)
        subq(3, 0, 1)
        for j in (1, 2, 3):
            wwait(j, 0)
            wwait(j, 1)
            for r in range(_RT):
                sub(r, j)

        oc = pltpu.make_async_copy(
            stage.at[0],
            out_hbm.at[pl.ds(my * m_per, m_per), :],
            out_sems.at[0],
        )
        oc.start()
        out_copies = [oc, None]

        slot = 1
        for j in (1, 2, 0):
            src = (my - offs[j]) % N_DEV
            if out_copies[slot] is not None:
                out_copies[slot].wait()
            for r in range(_RT):
                rows = pl.ds(r * rt, rt)
                if j == 0:
                    for h in range(_WH):
                        cols = pl.ds(h * wh, wh)
                        rdmas[(j, r, h)].wait_recv()
                        stage[slot, rows, cols] = recv_buf[
                            j, rows, cols
                        ].astype(jnp.float32)
                else:
                    rdmas[(j, r, 0)].wait_recv()
                    stage[slot, rows, :] = recv_buf[j, rows, :].astype(
                        jnp.float32
                    )
            oc = pltpu.make_async_copy(
                stage.at[slot],
                out_hbm.at[pl.ds(src * m_per, m_per), :],
                out_sems.at[slot],
            )
            oc.start()
            out_copies[slot] = oc
            slot ^= 1

        out_copies[0].wait()
        out_copies[1].wait()
        for rd in rdmas.values():
            rd.wait_send()

    out_shape = jax.ShapeDtypeStruct((N_DEV * m_per, n_per), jnp.float32)
    return pl.pallas_call(
        body,
        out_shape=out_shape,
        in_specs=[
            pl.BlockSpec(memory_space=pl.ANY),
            pl.BlockSpec(memory_space=pl.ANY),
        ],
        out_specs=pl.BlockSpec(memory_space=pl.ANY),
        scratch_shapes=[
            pltpu.VMEM((2, rt, k), jnp.float32),
            pltpu.VMEM((m_per, k), jnp.bfloat16),
            pltpu.VMEM((2, k, wh), jnp.float32),
            pltpu.VMEM((4, k, n_per), jnp.bfloat16),
            pltpu.VMEM((3, m_per, n_per), jnp.bfloat16),
            pltpu.VMEM((3, m_per, n_per), jnp.bfloat16),
            pltpu.VMEM((2, m_per, n_per), jnp.float32),
            pltpu.SemaphoreType.DMA((2,)),
            pltpu.SemaphoreType.DMA((2,)),
            pltpu.SemaphoreType.DMA((2,)),
            pltpu.SemaphoreType.DMA((3, _RT, _WH)),
            pltpu.SemaphoreType.DMA((3, _RT, _WH)),
        ],
        compiler_params=pltpu.CompilerParams(
            collective_id=0, vmem_limit_bytes=100 * 1024 * 1024
        ),
    )(x, w_mat)
